# baseline (device time: 21826 ns/iter reference)
import jax
import jax.numpy as jnp
from jax import lax
from jax.experimental import pallas as pl
from jax.experimental.pallas import tpu as pltpu

N_DEV = 8
B = 128
D = 128
H = 2048 // N_DEV
ROWS = B // N_DEV


def kernel(x, Win0, Wout0, Win1, Wout1, Win2, Wout2):
    xp = jnp.pad(x.astype(jnp.bfloat16), ((0, 0), (0, H - D)))
    winsp = jnp.concatenate([Win0, Win1, Win2], 0).astype(jnp.bfloat16)
    a_pack = jnp.concatenate([xp, winsp], 0)
    b_pack = jnp.concatenate([Wout0, Wout1, Wout2], 0).astype(jnp.bfloat16)
    WIN0 = B

    def body(a_ref, b_ref, out_ref,
             acc_ref, rs_ref, part_ref, send_sems, recv_sems):
        me = lax.axis_index("i")

        def layer_partial(xv, l):
            h = jnp.dot(xv, a_ref[WIN0 + l * D: WIN0 + (l + 1) * D, :],
                        preferred_element_type=jnp.float32)
            h = jnp.maximum(h, 0.0).astype(jnp.bfloat16)
            return jnp.dot(h, b_ref[l * H: (l + 1) * H, :],
                           preferred_element_type=jnp.float32)

        xv = a_ref[:B, :D]
        part = layer_partial(xv, 0)
        acc_ref[0, 0, :, :] = part.astype(jnp.bfloat16)

        barrier_sem = pltpu.get_barrier_semaphore()
        for off in range(1, N_DEV):
            pl.semaphore_signal(
                barrier_sem, inc=1,
                device_id=((me + off) % N_DEV,),
                device_id_type=pl.DeviceIdType.MESH,
            )
        pl.semaphore_wait(barrier_sem, N_DEV - 1)

        for l in range(2):
            buf = l % 2
            if l > 0:
                part = layer_partial(xv, l)
                acc_ref[buf, 0, :, :] = part.astype(jnp.bfloat16)
            rdmas = []
            for off in range(1, N_DEV):
                tgt = (me + off) % N_DEV
                rdma = pltpu.make_async_remote_copy(
                    src_ref=acc_ref.at[buf, 0],
                    dst_ref=acc_ref.at[buf, off],
                    send_sem=send_sems.at[l, off],
                    recv_sem=recv_sems.at[l, off],
                    device_id=(tgt,),
                    device_id_type=pl.DeviceIdType.MESH,
                )
                rdma.start()
                rdmas.append(rdma)
            xsum = part
            for off, rdma in zip(range(1, N_DEV), rdmas):
                rdma.wait_recv()
                xsum = xsum + acc_ref[buf, off, :, :].astype(jnp.float32)
            xv = xsum.astype(jnp.bfloat16)
            for rdma in rdmas:
                rdma.wait_send()

        part = layer_partial(xv, 2)
        part_ref[:, :] = part.astype(jnp.bfloat16)
        rdmas = []
        for off in range(1, N_DEV):
            tgt = (me + off) % N_DEV
            rdma = pltpu.make_async_remote_copy(
                src_ref=part_ref.at[pl.ds(tgt * ROWS, ROWS), :],
                dst_ref=rs_ref.at[off],
                send_sem=send_sems.at[2, off],
                recv_sem=recv_sems.at[2, off],
                device_id=(tgt,),
                device_id_type=pl.DeviceIdType.MESH,
            )
            rdma.start()
            rdmas.append(rdma)
        total = part_ref[pl.ds(me * ROWS, ROWS), :].astype(jnp.float32)
        for off, rdma in zip(range(1, N_DEV), rdmas):
            rdma.wait_recv()
            total = total + rs_ref[off, :, :].astype(jnp.float32)
        out_ref[:, :] = total.astype(jnp.bfloat16)
        for rdma in rdmas:
            rdma.wait_send()

    return pl.pallas_call(
        body,
        out_shape=jax.ShapeDtypeStruct((ROWS, D), jnp.bfloat16),
        in_specs=[pl.BlockSpec(memory_space=pltpu.VMEM)] * 2,
        out_specs=pl.BlockSpec(memory_space=pltpu.VMEM),
        scratch_shapes=[
            pltpu.VMEM((2, N_DEV, B, D), jnp.bfloat16),
            pltpu.VMEM((N_DEV, ROWS, D), jnp.bfloat16),
            pltpu.VMEM((B, D), jnp.bfloat16),
            pltpu.SemaphoreType.DMA((3, N_DEV)),
            pltpu.SemaphoreType.DMA((3, N_DEV)),
        ],
        compiler_params=pltpu.CompilerParams(collective_id=0),
    )(a_pack, b_pack)


# device time: 20078 ns/iter; 1.0871x vs baseline; 1.0871x over previous
import jax
import jax.numpy as jnp
from jax import lax
from jax.experimental import pallas as pl
from jax.experimental.pallas import tpu as pltpu

N_DEV = 8
B = 128
D = 128
H = 2048 // N_DEV
ROWS = B // N_DEV


def kernel(x, Win0, Wout0, Win1, Wout1, Win2, Wout2):
    wins = jnp.stack([Win0, Win1, Win2]).astype(jnp.bfloat16)
    wouts = jnp.stack([Wout0, Wout1, Wout2]).astype(jnp.bfloat16)

    def body(x_ref, wins_ref, wouts_ref, out_ref,
             acc_ref, rs_ref, part_ref, send_sems, recv_sems):
        me = lax.axis_index("i")

        def layer_partial(xv, l):
            h = jnp.dot(xv, wins_ref[l, :, :],
                        preferred_element_type=jnp.float32)
            h = jnp.maximum(h, 0.0).astype(jnp.bfloat16)
            return jnp.dot(h, wouts_ref[l, :, :],
                           preferred_element_type=jnp.float32)

        xv = x_ref[:, :].astype(jnp.bfloat16)
        part = layer_partial(xv, 0)
        acc_ref[0, 0, :, :] = part.astype(jnp.bfloat16)

        barrier_sem = pltpu.get_barrier_semaphore()
        for off in range(1, N_DEV):
            pl.semaphore_signal(
                barrier_sem, inc=1,
                device_id=((me + off) % N_DEV,),
                device_id_type=pl.DeviceIdType.MESH,
            )
        pl.semaphore_wait(barrier_sem, N_DEV - 1)

        for l in range(2):
            buf = l % 2
            if l > 0:
                part = layer_partial(xv, l)
                acc_ref[buf, 0, :, :] = part.astype(jnp.bfloat16)
            rdmas = []
            for off in range(1, N_DEV):
                tgt = (me + off) % N_DEV
                rdma = pltpu.make_async_remote_copy(
                    src_ref=acc_ref.at[buf, 0],
                    dst_ref=acc_ref.at[buf, off],
                    send_sem=send_sems.at[l, off],
                    recv_sem=recv_sems.at[l, off],
                    device_id=(tgt,),
                    device_id_type=pl.DeviceIdType.MESH,
                )
                rdma.start()
                rdmas.append(rdma)
            xsum = part
            for off, rdma in zip(range(1, N_DEV), rdmas):
                rdma.wait_recv()
                xsum = xsum + acc_ref[buf, off, :, :].astype(jnp.float32)
            xv = xsum.astype(jnp.bfloat16)
            for rdma in rdmas:
                rdma.wait_send()

        part = layer_partial(xv, 2)
        part_ref[:, :] = part.astype(jnp.bfloat16)
        rdmas = []
        for off in range(1, N_DEV):
            tgt = (me + off) % N_DEV
            rdma = pltpu.make_async_remote_copy(
                src_ref=part_ref.at[pl.ds(tgt * ROWS, ROWS), :],
                dst_ref=rs_ref.at[off],
                send_sem=send_sems.at[2, off],
                recv_sem=recv_sems.at[2, off],
                device_id=(tgt,),
                device_id_type=pl.DeviceIdType.MESH,
            )
            rdma.start()
            rdmas.append(rdma)
        total = part_ref[pl.ds(me * ROWS, ROWS), :].astype(jnp.float32)
        for off, rdma in zip(range(1, N_DEV), rdmas):
            rdma.wait_recv()
            total = total + rs_ref[off, :, :].astype(jnp.float32)
        out_ref[:, :] = total
        for rdma in rdmas:
            rdma.wait_send()

    return pl.pallas_call(
        body,
        out_shape=jax.ShapeDtypeStruct((ROWS, D), jnp.float32),
        in_specs=[pl.BlockSpec(memory_space=pltpu.VMEM)] * 3,
        out_specs=pl.BlockSpec(memory_space=pltpu.VMEM),
        scratch_shapes=[
            pltpu.VMEM((2, N_DEV, B, D), jnp.bfloat16),
            pltpu.VMEM((N_DEV, ROWS, D), jnp.bfloat16),
            pltpu.VMEM((B, D), jnp.bfloat16),
            pltpu.SemaphoreType.DMA((3, N_DEV)),
            pltpu.SemaphoreType.DMA((3, N_DEV)),
        ],
        compiler_params=pltpu.CompilerParams(collective_id=0),
    )(x, wins, wouts)


# device time: 19901 ns/iter; 1.0967x vs baseline; 1.0089x over previous
import jax
import jax.numpy as jnp
from jax import lax
from jax.experimental import pallas as pl
from jax.experimental.pallas import tpu as pltpu

N_DEV = 8
B = 128
D = 128
H = 2048 // N_DEV
ROWS = B // N_DEV


def kernel(x, Win0, Wout0, Win1, Wout1, Win2, Wout2):
    wins = jnp.stack([Win0, Win1, Win2]).astype(jnp.bfloat16)
    wouts = jnp.stack([Wout0, Wout1, Wout2]).astype(jnp.bfloat16)

    def body(x_ref, wins_ref, wouts_ref, out_ref,
             acc_ref, rs_ref, part_ref, send_sems, recv_sems):
        me = lax.axis_index("i")

        def layer_partial(xv, l):
            h = jnp.dot(xv, wins_ref[l, :, :],
                        preferred_element_type=jnp.float32)
            h = jnp.maximum(h, 0.0).astype(jnp.bfloat16)
            return jnp.dot(h, wouts_ref[l, :, :],
                           preferred_element_type=jnp.float32)

        barrier_sem = pltpu.get_barrier_semaphore()
        for off in range(1, N_DEV):
            pl.semaphore_signal(
                barrier_sem, inc=1,
                device_id=((me + off) % N_DEV,),
                device_id_type=pl.DeviceIdType.MESH,
            )
        xv = x_ref[:, :].astype(jnp.bfloat16)
        part = layer_partial(xv, 0)
        acc_ref[0, 0, :, :] = part.astype(jnp.bfloat16)
        pl.semaphore_wait(barrier_sem, N_DEV - 1)

        for l in range(2):
            buf = l % 2
            rdmas = []
            for off in range(1, N_DEV):
                tgt = (me + off) % N_DEV
                rdma = pltpu.make_async_remote_copy(
                    src_ref=acc_ref.at[buf, 0],
                    dst_ref=acc_ref.at[buf, off],
                    send_sem=send_sems.at[l, off],
                    recv_sem=recv_sems.at[l, off],
                    device_id=(tgt,),
                    device_id_type=pl.DeviceIdType.MESH,
                )
                rdma.start()
                rdmas.append(rdma)
            hacc = jnp.dot(acc_ref[buf, 0, :, :], wins_ref[l + 1, :, :],
                           preferred_element_type=jnp.float32)
            for off, rdma in zip(range(1, N_DEV), rdmas):
                rdma.wait_recv()
                hacc = hacc + jnp.dot(
                    acc_ref[buf, off, :, :], wins_ref[l + 1, :, :],
                    preferred_element_type=jnp.float32)
            h = jnp.maximum(hacc, 0.0).astype(jnp.bfloat16)
            part = jnp.dot(h, wouts_ref[l + 1, :, :],
                           preferred_element_type=jnp.float32)
            if l == 0:
                acc_ref[1, 0, :, :] = part.astype(jnp.bfloat16)
            for rdma in rdmas:
                rdma.wait_send()

        part_ref[:, :] = part.astype(jnp.bfloat16)
        rdmas = []
        for off in range(1, N_DEV):
            tgt = (me + off) % N_DEV
            rdma = pltpu.make_async_remote_copy(
                src_ref=part_ref.at[pl.ds(tgt * ROWS, ROWS), :],
                dst_ref=rs_ref.at[off],
                send_sem=send_sems.at[2, off],
                recv_sem=recv_sems.at[2, off],
                device_id=(tgt,),
                device_id_type=pl.DeviceIdType.MESH,
            )
            rdma.start()
            rdmas.append(rdma)
        total = part_ref[pl.ds(me * ROWS, ROWS), :].astype(jnp.float32)
        for off, rdma in zip(range(1, N_DEV), rdmas):
            rdma.wait_recv()
            total = total + rs_ref[off, :, :].astype(jnp.float32)
        out_ref[:, :] = total
        for rdma in rdmas:
            rdma.wait_send()

    return pl.pallas_call(
        body,
        out_shape=jax.ShapeDtypeStruct((ROWS, D), jnp.float32),
        in_specs=[pl.BlockSpec(memory_space=pltpu.VMEM)] * 3,
        out_specs=pl.BlockSpec(memory_space=pltpu.VMEM),
        scratch_shapes=[
            pltpu.VMEM((2, N_DEV, B, D), jnp.bfloat16),
            pltpu.VMEM((N_DEV, ROWS, D), jnp.bfloat16),
            pltpu.VMEM((B, D), jnp.bfloat16),
            pltpu.SemaphoreType.DMA((3, N_DEV)),
            pltpu.SemaphoreType.DMA((3, N_DEV)),
        ],
        compiler_params=pltpu.CompilerParams(collective_id=0),
    )(x, wins, wouts)
